# baseline (speedup 1.0000x reference)
import functools

import jax
import jax.numpy as jnp
import numpy as np
from jax import lax
from jax.experimental import pallas as pl
from jax.experimental.pallas import tpu as pltpu
from jax.experimental.pallas import tpu_sc as plsc

N, C, H, W = 4, 192, 56, 56
G = 4
Cg = C // G
KS = 3
P = KS * KS
KOUT = int(np.ceil(G * P * 3 / 8) * 8)
L = H * W
NL = N * L
GP = G * P
WL = 4 * GP
HP = H + 4
WP = W + 4

NW = 32
PX_PER_W = NL // NW
CH = 7
NCHUNK = PX_PER_W // CH

_gp_g = np.repeat(np.arange(G), P)
_gp_p = np.tile(np.arange(P), G)
_PERM = np.concatenate([
    _gp_g * 27 + 2 * _gp_p,
    _gp_g * 27 + 2 * _gp_p + 1,
    _gp_g * 27 + 18 + _gp_p,
    np.arange(G * P * 3, KOUT),
]).astype(np.int32)

_KYV = (_gp_p // KS).astype(np.float32)
_KXV = (_gp_p % KS).astype(np.float32)
_GOFF = _gp_g.astype(np.int32)


def _prep_body(xp_ref, w_ref, b_ref, a_ref, b2_ref, kf_ref, gb_ref,
               tbl_ref, idx_ref, wcat_ref):
    n = pl.program_id(0)
    xs = xp_ref[0, 1:57, 1:57, :].reshape(L, C)
    y = lax.dot_general(xs, w_ref[...], (((1,), (1,)), ((), ())),
                        preferred_element_type=jnp.float32) + b_ref[...]
    tbl_ref[...] = jnp.zeros_like(tbl_ref)
    tbl_ref[0, 2:58, 2:58, :] = y.reshape(H, W, C)

    acc = jnp.zeros((L, KOUT), jnp.float32)
    for j in range(P):
        ky, kx = j // KS, j % KS
        xsj = xp_ref[0, ky:ky + H, kx:kx + W, :].reshape(L, C)
        acc += lax.dot_general(xsj, a_ref[j], (((1,), (1,)), ((), ())),
                               preferred_element_type=jnp.float32)
    om = (acc + b2_ref[...]).reshape(H, W, KOUT)
    dx = om[..., 0:GP]
    dy = om[..., GP:2 * GP]
    msk = om[..., 2 * GP:3 * GP]
    hh = lax.broadcasted_iota(jnp.int32, (H, W, GP), 0).astype(jnp.float32)
    ww = lax.broadcasted_iota(jnp.int32, (H, W, GP), 1).astype(jnp.float32)
    kyv = kf_ref[0][None, None, :]
    kxv = kf_ref[1][None, None, :]
    sy = hh - 1.0 + kyv + dy
    sx = ww - 1.0 + kxv + dx
    y0 = jnp.floor(sy)
    x0 = jnp.floor(sx)
    wy1 = sy - y0
    wx1 = sx - x0
    wy0 = 1.0 - wy1
    wx0 = 1.0 - wx1
    base = (n * (HP * WP * G) + gb_ref[0])[None, None, :]

    def cidx(ycf, xcf):
        uy = jnp.clip(ycf + 2.0, 0.0, 59.0).astype(jnp.int32)
        ux = jnp.clip(xcf + 2.0, 0.0, 59.0).astype(jnp.int32)
        return base + (uy * WP + ux) * G

    idx_ref[...] = jnp.concatenate(
        [cidx(y0, x0), cidx(y0, x0 + 1.0),
         cidx(y0 + 1.0, x0), cidx(y0 + 1.0, x0 + 1.0)],
        axis=-1).reshape(1, L, WL)
    wcat_ref[...] = jnp.concatenate(
        [msk * wy0 * wx0, msk * wy0 * wx1, msk * wy1 * wx0, msk * wy1 * wx1],
        axis=-1).reshape(1, L, WL)


def _prep(xpad, W_in, b_in, A9, b2):
    tbl, idx, wcat = pl.pallas_call(
        _prep_body,
        grid=(N,),
        in_specs=[
            pl.BlockSpec((1, H + 2, W + 2, C), lambda n: (n, 0, 0, 0)),
            pl.BlockSpec((C, C), lambda n: (0, 0)),
            pl.BlockSpec((1, C), lambda n: (0, 0)),
            pl.BlockSpec((P, KOUT, C), lambda n: (0, 0, 0)),
            pl.BlockSpec((1, KOUT), lambda n: (0, 0)),
            pl.BlockSpec((2, GP), lambda n: (0, 0)),
            pl.BlockSpec((1, GP), lambda n: (0, 0)),
        ],
        out_specs=[
            pl.BlockSpec((1, HP, WP, C), lambda n: (n, 0, 0, 0)),
            pl.BlockSpec((1, L, WL), lambda n: (n, 0, 0)),
            pl.BlockSpec((1, L, WL), lambda n: (n, 0, 0)),
        ],
        out_shape=[
            jax.ShapeDtypeStruct((N, HP, WP, C), jnp.float32),
            jax.ShapeDtypeStruct((N, L, WL), jnp.int32),
            jax.ShapeDtypeStruct((N, L, WL), jnp.float32),
        ],
    )(xpad, W_in, b_in.reshape(1, C), A9, b2,
      jnp.asarray(np.stack([_KYV, _KXV])), jnp.asarray(_GOFF).reshape(1, GP))
    return (tbl.reshape(N * HP * WP * G, Cg), idx.reshape(NL * WL),
            wcat.reshape(NL * WL))


@functools.lru_cache(maxsize=1)
def _make_sc_sample():
    mesh = plsc.VectorSubcoreMesh(core_axis_name="c", subcore_axis_name="s")
    return functools.partial(
        pl.kernel,
        out_type=jax.ShapeDtypeStruct((NL, C), jnp.float32),
        mesh=mesh,
        scratch_types=(
            [pltpu.VMEM((CH * WL,), jnp.int32) for _ in range(2)]
            + [pltpu.VMEM((CH * WL,), jnp.float32) for _ in range(2)]
            + [pltpu.VMEM((CH * WL, Cg), jnp.float32) for _ in range(2)]
            + [pltpu.VMEM((CH, C), jnp.float32) for _ in range(2)]
            + [pltpu.SemaphoreType.DMA for _ in range(6)]
        ),
        compiler_params=pltpu.CompilerParams(needs_layout_passes=False,
                                             use_tc_tiling_on_sc=False),
    )(_sc_sample_body)


def _sc_sample_body(tbl, idx, wcat, out_hbm,
                    iva, ivb, wva, wvb, rva, rvb, ova, ovb,
                    sem_ia, sem_ib, sem_ra, sem_rb, sem_oa, sem_ob):
    wid = lax.axis_index("s") * 2 + lax.axis_index("c")
    base_px = wid * PX_PER_W
    LAST = NCHUNK - 1

    def idx_descs(ck, ivx, wvx, sem):
        e0 = (base_px + ck * CH) * WL
        return [pltpu.make_async_copy(idx.at[pl.ds(e0, CH * WL)], ivx, sem),
                pltpu.make_async_copy(wcat.at[pl.ds(e0, CH * WL)], wvx, sem)]

    def gather_descs(ivx, rvx, sem):
        return [pltpu.make_async_copy(tbl.at[ivx], rvx, sem)]

    def issue(descs):
        for d in descs:
            d.start()

    def drain(descs):
        for d in descs:
            d.wait()

    def out_desc(ck, ovx, sem):
        return [pltpu.make_async_copy(
            ovx, out_hbm.at[pl.ds(base_px + ck * CH, CH)], sem)]

    def compute(ck, rvx, wvx, ovx, sem_o):
        @pl.when(ck >= 2)
        def _wait_prev():
            drain(out_desc(ck, ovx, sem_o))

        @plsc.parallel_loop(0, CH, 1)
        def px_body(px):
            pxb = px * WL
            accs = [jnp.zeros((16,), jnp.float32) for _ in range(12)]
            for j in range(GP):
                g = j // P
                for c in range(4):
                    q = pxb + c * GP + j
                    wsp = plsc.load_gather(wvx, [jnp.full((16,), q, jnp.int32)])
                    for k in range(3):
                        row = rvx[q, pl.ds(k * 16, 16)]
                        accs[g * 3 + k] = accs[g * 3 + k] + wsp * row
            for t in range(12):
                ovx[px, pl.ds(t * 16, 16)] = accs[t]

        issue(out_desc(ck, ovx, sem_o))

    def phase(c_nxt1, c_nxt2, ivc, wvc, rvc, sem_ic, sem_rc,
              ivo, wvo, rvo, sem_io, sem_ro, c_cur, ovc, sem_oc):
        drain(idx_descs(c_nxt1, ivo, wvo, sem_io))
        issue(gather_descs(ivo, rvo, sem_ro))
        drain(gather_descs(ivc, rvc, sem_rc))
        issue(idx_descs(c_nxt2, ivc, wvc, sem_ic)[:1])
        compute(c_cur, rvc, wvc, ovc, sem_oc)
        issue(idx_descs(c_nxt2, ivc, wvc, sem_ic)[1:])

    issue(idx_descs(0, iva, wva, sem_ia))
    issue(idx_descs(1, ivb, wvb, sem_ib))
    drain(idx_descs(0, iva, wva, sem_ia))
    issue(gather_descs(iva, rva, sem_ra))

    def pair_body(i, carry):
        cA = 2 * i
        cB = cA + 1
        nA2 = jnp.minimum(cA + 2, LAST)
        nB1 = jnp.minimum(cB + 1, LAST)
        nB2 = jnp.minimum(cB + 2, LAST)
        phase(cB, nA2, iva, wva, rva, sem_ia, sem_ra,
              ivb, wvb, rvb, sem_ib, sem_rb, cA, ova, sem_oa)
        phase(nB1, nB2, ivb, wvb, rvb, sem_ib, sem_rb,
              iva, wva, rva, sem_ia, sem_ra, cB, ovb, sem_ob)
        return carry

    lax.fori_loop(0, NCHUNK // 2, pair_body, 0)
    drain(gather_descs(iva, rva, sem_ra))
    drain(idx_descs(LAST, ivb, wvb, sem_ib))
    drain(out_desc(LAST - 1, ova, sem_oa))
    drain(out_desc(LAST, ovb, sem_ob))


def _proj_out_body(x_ref, w_ref, b_ref, o_ref):
    o_ref[0] = lax.dot_general(w_ref[...], x_ref[0].astype(jnp.bfloat16),
                               (((1,), (1,)), ((), ())),
                               preferred_element_type=jnp.float32) + b_ref[...]


def _proj_out(res, W_out, b_out):
    return pl.pallas_call(
        _proj_out_body,
        grid=(N,),
        in_specs=[
            pl.BlockSpec((1, L, C), lambda n: (n, 0, 0)),
            pl.BlockSpec((C, C), lambda n: (0, 0)),
            pl.BlockSpec((C, 1), lambda n: (0, 0)),
        ],
        out_specs=pl.BlockSpec((1, C, L), lambda n: (n, 0, 0)),
        out_shape=jax.ShapeDtypeStruct((N, C, L), jnp.float32),
    )(res, W_out.astype(jnp.bfloat16), b_out.reshape(C, 1))


def kernel(input, W_in, b_in, dw_w, dw_b, pw_w, pw_b, W_out, b_out):
    x_nhwc = input.transpose(0, 2, 3, 1)
    xpad = jnp.pad(x_nhwc, ((0, 0), (1, 1), (1, 1), (0, 0))).astype(jnp.bfloat16)

    pw_p = pw_w[_PERM]
    b2 = (pw_b[_PERM] + pw_p @ dw_b).reshape(1, KOUT)
    dwf = dw_w.reshape(C, P)
    A9 = (pw_p[None, :, :] * dwf.T[:, None, :]).astype(jnp.bfloat16)

    tbl, idx, wcat = _prep(xpad, W_in.astype(jnp.bfloat16), b_in, A9, b2)
    res = _make_sc_sample()(tbl, idx, wcat)
    out2 = _proj_out(res.reshape(N, L, C), W_out, b_out)
    return out2.reshape(N, C, H, W)


# speedup vs baseline: 1.0125x; 1.0125x over previous
import functools

import jax
import jax.numpy as jnp
import numpy as np
from jax import lax
from jax.experimental import pallas as pl
from jax.experimental.pallas import tpu as pltpu
from jax.experimental.pallas import tpu_sc as plsc

N, C, H, W = 4, 192, 56, 56
G = 4
Cg = C // G
KS = 3
P = KS * KS
KOUT = int(np.ceil(G * P * 3 / 8) * 8)
L = H * W
NL = N * L
GP = G * P
WL = 4 * GP
HP = H + 4
WP = W + 4

NW = 32
PX_PER_W = NL // NW
CH = 7
NCHUNK = PX_PER_W // CH

_gp_g = np.repeat(np.arange(G), P)
_gp_p = np.tile(np.arange(P), G)
_PERM = np.concatenate([
    _gp_g * 27 + 2 * _gp_p,
    _gp_g * 27 + 2 * _gp_p + 1,
    _gp_g * 27 + 18 + _gp_p,
    np.arange(G * P * 3, KOUT),
]).astype(np.int32)

_KYV = (_gp_p // KS).astype(np.float32)
_KXV = (_gp_p % KS).astype(np.float32)
_GOFF = _gp_g.astype(np.int32)


def _prep_body(xp_ref, w_ref, b_ref, a_ref, b2_ref, kf_ref, gb_ref,
               tbl_ref, idx_ref, wcat_ref):
    n = pl.program_id(0)
    xs = xp_ref[0, 1:57, 1:57, :].reshape(L, C)
    y = lax.dot_general(xs, w_ref[...], (((1,), (1,)), ((), ())),
                        preferred_element_type=jnp.float32) + b_ref[...]
    tbl_ref[...] = jnp.zeros_like(tbl_ref)
    tbl_ref[0, 2:58, 2:58, :] = y.reshape(H, W, C)

    acc = jnp.zeros((L, KOUT), jnp.float32)
    for j in range(P):
        ky, kx = j // KS, j % KS
        xsj = xp_ref[0, ky:ky + H, kx:kx + W, :].reshape(L, C)
        acc += lax.dot_general(xsj, a_ref[j], (((1,), (1,)), ((), ())),
                               preferred_element_type=jnp.float32)
    om = (acc + b2_ref[...]).reshape(H, W, KOUT)
    dx = om[..., 0:GP]
    dy = om[..., GP:2 * GP]
    msk = om[..., 2 * GP:3 * GP]
    hh = lax.broadcasted_iota(jnp.int32, (H, W, GP), 0).astype(jnp.float32)
    ww = lax.broadcasted_iota(jnp.int32, (H, W, GP), 1).astype(jnp.float32)
    kyv = kf_ref[0][None, None, :]
    kxv = kf_ref[1][None, None, :]
    sy = hh - 1.0 + kyv + dy
    sx = ww - 1.0 + kxv + dx
    y0 = jnp.floor(sy)
    x0 = jnp.floor(sx)
    wy1 = sy - y0
    wx1 = sx - x0
    wy0 = 1.0 - wy1
    wx0 = 1.0 - wx1
    base = (n * (HP * WP * G) + gb_ref[0])[None, None, :]

    def cidx(ycf, xcf):
        uy = jnp.clip(ycf + 2.0, 0.0, 59.0).astype(jnp.int32)
        ux = jnp.clip(xcf + 2.0, 0.0, 59.0).astype(jnp.int32)
        return base + (uy * WP + ux) * G

    idx_ref[...] = jnp.concatenate(
        [cidx(y0, x0), cidx(y0, x0 + 1.0),
         cidx(y0 + 1.0, x0), cidx(y0 + 1.0, x0 + 1.0)],
        axis=-1).reshape(1, L, WL)
    wcat_ref[...] = jnp.concatenate(
        [msk * wy0 * wx0, msk * wy0 * wx1, msk * wy1 * wx0, msk * wy1 * wx1],
        axis=-1).reshape(1, L, WL)


def _prep(xpad, W_in, b_in, A9, b2):
    tbl, idx, wcat = pl.pallas_call(
        _prep_body,
        grid=(N,),
        in_specs=[
            pl.BlockSpec((1, H + 2, W + 2, C), lambda n: (n, 0, 0, 0)),
            pl.BlockSpec((C, C), lambda n: (0, 0)),
            pl.BlockSpec((1, C), lambda n: (0, 0)),
            pl.BlockSpec((P, KOUT, C), lambda n: (0, 0, 0)),
            pl.BlockSpec((1, KOUT), lambda n: (0, 0)),
            pl.BlockSpec((2, GP), lambda n: (0, 0)),
            pl.BlockSpec((1, GP), lambda n: (0, 0)),
        ],
        out_specs=[
            pl.BlockSpec((1, HP, WP, C), lambda n: (n, 0, 0, 0)),
            pl.BlockSpec((1, L, WL), lambda n: (n, 0, 0)),
            pl.BlockSpec((1, L, WL), lambda n: (n, 0, 0)),
        ],
        out_shape=[
            jax.ShapeDtypeStruct((N, HP, WP, C), jnp.float32),
            jax.ShapeDtypeStruct((N, L, WL), jnp.int32),
            jax.ShapeDtypeStruct((N, L, WL), jnp.float32),
        ],
    )(xpad, W_in, b_in.reshape(1, C), A9, b2,
      jnp.asarray(np.stack([_KYV, _KXV])), jnp.asarray(_GOFF).reshape(1, GP))
    return (tbl.reshape(N * HP * WP * G, Cg), idx.reshape(NL * WL),
            wcat.reshape(NL * WL))


@functools.lru_cache(maxsize=1)
def _make_sc_sample():
    mesh = plsc.VectorSubcoreMesh(core_axis_name="c", subcore_axis_name="s")
    return functools.partial(
        pl.kernel,
        out_type=jax.ShapeDtypeStruct((NL, C), jnp.float32),
        mesh=mesh,
        scratch_types=(
            [pltpu.VMEM((CH * WL,), jnp.int32) for _ in range(2)]
            + [pltpu.VMEM((CH * WL,), jnp.float32) for _ in range(2)]
            + [pltpu.VMEM((CH * WL, Cg), jnp.float32) for _ in range(2)]
            + [pltpu.VMEM((CH, C), jnp.float32) for _ in range(2)]
            + [pltpu.SemaphoreType.DMA for _ in range(6)]
        ),
        compiler_params=pltpu.CompilerParams(needs_layout_passes=False,
                                             use_tc_tiling_on_sc=False),
    )(_sc_sample_body)


def _sc_sample_body(tbl, idx, wcat, out_hbm,
                    iva, ivb, wva, wvb, rva, rvb, ova, ovb,
                    sem_ia, sem_ib, sem_ra, sem_rb, sem_oa, sem_ob):
    wid = lax.axis_index("s") * 2 + lax.axis_index("c")
    base_px = wid * PX_PER_W
    LAST = NCHUNK - 1

    def idx_descs(ck, ivx, wvx, sem):
        e0 = (base_px + ck * CH) * WL
        return [pltpu.make_async_copy(idx.at[pl.ds(e0, CH * WL)], ivx, sem),
                pltpu.make_async_copy(wcat.at[pl.ds(e0, CH * WL)], wvx, sem)]

    def gather_descs(ivx, rvx, sem):
        return [pltpu.make_async_copy(tbl.at[ivx], rvx, sem)]

    def issue(descs):
        for d in descs:
            d.start()

    def drain(descs):
        for d in descs:
            d.wait()

    def out_desc(ck, ovx, sem):
        return [pltpu.make_async_copy(
            ovx, out_hbm.at[pl.ds(base_px + ck * CH, CH)], sem)]

    def compute(ck, rvx, wvx, ovx, sem_o):
        @pl.when(ck >= 2)
        def _wait_prev():
            drain(out_desc(ck, ovx, sem_o))

        @plsc.parallel_loop(0, CH, 1)
        def px_body(px):
            pxb = px * WL
            accs = [jnp.zeros((16,), jnp.float32) for _ in range(12)]
            for j in range(GP):
                g = j // P
                for c in range(4):
                    q = pxb + c * GP + j
                    wsp = plsc.load_gather(wvx, [jnp.full((16,), q, jnp.int32)])
                    for k in range(3):
                        row = rvx[q, pl.ds(k * 16, 16)]
                        accs[g * 3 + k] = accs[g * 3 + k] + wsp * row
            for t in range(12):
                ovx[px, pl.ds(t * 16, 16)] = accs[t]

        issue(out_desc(ck, ovx, sem_o))

    def phase(c_nxt1, c_nxt2, ivc, wvc, rvc, sem_ic, sem_rc,
              ivo, wvo, rvo, sem_io, sem_ro, c_cur, ovc, sem_oc):
        drain(idx_descs(c_nxt1, ivo, wvo, sem_io))
        issue(gather_descs(ivo, rvo, sem_ro))
        drain(gather_descs(ivc, rvc, sem_rc))
        issue(idx_descs(c_nxt2, ivc, wvc, sem_ic)[:1])
        compute(c_cur, rvc, wvc, ovc, sem_oc)
        issue(idx_descs(c_nxt2, ivc, wvc, sem_ic)[1:])

    issue(idx_descs(0, iva, wva, sem_ia))
    issue(idx_descs(1, ivb, wvb, sem_ib))
    drain(idx_descs(0, iva, wva, sem_ia))
    issue(gather_descs(iva, rva, sem_ra))

    def pair_body(i, carry):
        cA = 2 * i
        cB = cA + 1
        nA2 = jnp.minimum(cA + 2, LAST)
        nB1 = jnp.minimum(cB + 1, LAST)
        nB2 = jnp.minimum(cB + 2, LAST)
        phase(cB, nA2, iva, wva, rva, sem_ia, sem_ra,
              ivb, wvb, rvb, sem_ib, sem_rb, cA, ova, sem_oa)
        phase(nB1, nB2, ivb, wvb, rvb, sem_ib, sem_rb,
              iva, wva, rva, sem_ia, sem_ra, cB, ovb, sem_ob)
        return carry

    lax.fori_loop(0, NCHUNK // 2, pair_body, 0)
    drain(gather_descs(iva, rva, sem_ra))
    drain(idx_descs(LAST, ivb, wvb, sem_ib))
    drain(out_desc(LAST - 1, ova, sem_oa))
    drain(out_desc(LAST, ovb, sem_ob))


def _proj_out_body(x_ref, w_ref, b_ref, o_ref):
    o_ref[0] = lax.dot_general(w_ref[...], x_ref[0], (((1,), (1,)), ((), ())),
                               preferred_element_type=jnp.float32) + b_ref[...]


def _proj_out(res, W_out, b_out):
    return pl.pallas_call(
        _proj_out_body,
        grid=(N,),
        in_specs=[
            pl.BlockSpec((1, L, C), lambda n: (n, 0, 0)),
            pl.BlockSpec((C, C), lambda n: (0, 0)),
            pl.BlockSpec((C, 1), lambda n: (0, 0)),
        ],
        out_specs=pl.BlockSpec((1, C, L), lambda n: (n, 0, 0)),
        out_shape=jax.ShapeDtypeStruct((N, C, L), jnp.float32),
    )(res, W_out, b_out.reshape(C, 1))


def kernel(input, W_in, b_in, dw_w, dw_b, pw_w, pw_b, W_out, b_out):
    x_nhwc = input.transpose(0, 2, 3, 1)
    xpad = jnp.pad(x_nhwc, ((0, 0), (1, 1), (1, 1), (0, 0)))

    pw_p = pw_w[_PERM]
    b2 = (pw_b[_PERM] + pw_p @ dw_b).reshape(1, KOUT)
    dwf = dw_w.reshape(C, P)
    A9 = pw_p[None, :, :] * dwf.T[:, None, :]

    tbl, idx, wcat = _prep(xpad, W_in, b_in, A9, b2)
    res = _make_sc_sample()(tbl, idx, wcat)
    out2 = _proj_out(res.reshape(N, L, C), W_out, b_out)
    return out2.reshape(N, C, H, W)
